# Initial kernel scaffold; baseline (speedup 1.0000x reference)
import jax
import jax.numpy as jnp
from jax.experimental import pallas as pl


def kernel(indices, tables):
    raise NotImplementedError("write your pallas kernel here")


# speedup vs baseline: 9.0440x; 9.0440x over previous
import functools

import jax
import jax.numpy as jnp
from jax import lax
from jax.experimental import pallas as pl
from jax.experimental.pallas import tpu as pltpu
from jax.experimental.pallas import tpu_sc as plsc

B = 16384
F = 26
V = 100000
D = 32
BF = B * F

NC, NS = 2, 16
NW = NC * NS
BPW = BF // NW
GSZ = 128
CHUNK = 1024
NG = CHUNK // GSZ
NCHUNK = BPW // CHUNK
IDX_ROWS = BF // GSZ


def _make_kernel():
    mesh = plsc.VectorSubcoreMesh(core_axis_name="c", subcore_axis_name="s")

    @functools.partial(
        pl.kernel,
        mesh=mesh,
        out_type=jax.ShapeDtypeStruct((BF, D), jnp.float32),
        compiler_params=pltpu.CompilerParams(use_tc_tiling_on_sc=False),
        scratch_types=[
            pltpu.VMEM((NG, GSZ), jnp.int32),
            pltpu.VMEM((CHUNK, D), jnp.float32),
            pltpu.SemaphoreType.DMA,
        ],
    )
    def gather_kernel(idx_hbm, table_hbm, out_hbm, idx_v, rows_v, sem):
        wid = lax.axis_index("s") * NC + lax.axis_index("c")
        row_base = wid * (BPW // GSZ)
        out_base = wid * BPW

        def chunk_body(c, carry):
            pltpu.sync_copy(idx_hbm.at[pl.ds(row_base + c * NG, NG)], idx_v)
            descs = [
                pltpu.async_copy(
                    table_hbm.at[idx_v.at[j]],
                    rows_v.at[pl.ds(j * GSZ, GSZ)],
                    sem,
                )
                for j in range(NG)
            ]
            for d in descs:
                d.wait()
            pltpu.sync_copy(
                rows_v, out_hbm.at[pl.ds(out_base + c * CHUNK, CHUNK)]
            )
            return carry

        lax.fori_loop(0, NCHUNK, chunk_body, 0)

    return gather_kernel


_gather = _make_kernel()


def kernel(indices, tables):
    flat_tables = tables.reshape(F * V, D)
    offsets = (jnp.arange(F, dtype=indices.dtype) * V)[None, :]
    flat_idx = (indices + offsets).reshape(IDX_ROWS, GSZ)
    out = _gather(flat_idx, flat_tables)
    return out.reshape(B, F * D)
